# baseline (speedup 1.0000x reference)
import functools

import jax
import jax.numpy as jnp
from jax import lax
from jax.experimental import pallas as pl
from jax.experimental.pallas import tpu as pltpu
from jax.experimental.pallas import tpu_sc as plsc

NU = 10000
NI = 10000
DIM = 128
NLAYER = 3
NE = 320000
NB = 16384

NCORE = 2
NSUB = 16
LANE = 16

CHUNK = 128
EPT = NE // NSUB
NCHUNKP = 160
EPTP = NCHUNKP * CHUNK
NPIPE = NCHUNKP // 2
NUP = 10240
RPT = NUP // NSUB

_sc_mesh = plsc.VectorSubcoreMesh(core_axis_name="c", subcore_axis_name="s")


def _sc_spmm_body(ecat, srcs, dsts, vals, out,
                  src0, src1, dstl0, dstl1, val0, val1, sdst0, sdst1,
                  rows0, rows1, accum,
                  isem0, isem1, gsem0, gsem1, ssem0, ssem1):
    cid = lax.axis_index("c")
    sid = lax.axis_index("s")
    row0 = sid * RPT
    zv = jnp.zeros((LANE,), jnp.float32)

    def scale(buf, valb):
        def grp(g, cc):
            base = g * LANE
            vv = valb[pl.ds(base, LANE)]
            for t in range(LANE):
                bv = vv[t]
                for j in range(DIM // LANE):
                    sl = pl.ds(j * LANE, LANE)
                    buf[base + t, sl] = buf[base + t, sl] * bv
            return cc

        lax.fori_loop(0, CHUNK // LANE, grp, 0)

    def copy_dst(dl, sd):
        for g in range(CHUNK // LANE):
            sl = pl.ds(g * LANE, LANE)
            sd[sl] = dl[sl]

    def run_phase(s, carry):
        p = cid * 2 + s
        ebase = (p * NSUB + sid) * EPTP

        def idx_load(c, sb, db, vb, sem):
            off = ebase + c * CHUNK
            pltpu.async_copy(srcs.at[pl.ds(off, CHUNK)], sb, sem)
            pltpu.async_copy(dsts.at[pl.ds(off, CHUNK)], db, sem)
            pltpu.async_copy(vals.at[pl.ds(off, CHUNK)], vb, sem)

        def idx_wait(c, sb, db, vb, sem):
            off = ebase + c * CHUNK
            pltpu.make_async_copy(srcs.at[pl.ds(off, CHUNK)], sb, sem).wait()
            pltpu.make_async_copy(dsts.at[pl.ds(off, CHUNK)], db, sem).wait()
            pltpu.make_async_copy(vals.at[pl.ds(off, CHUNK)], vb, sem).wait()

        def zrow(r, cc):
            for j in range(DIM // LANE):
                rows0[r, pl.ds(j * LANE, LANE)] = zv
            return cc

        lax.fori_loop(0, CHUNK, zrow, 0)
        for k in range(RPT // CHUNK):
            pltpu.sync_copy(rows0, accum.at[pl.ds(row0 + k * CHUNK, CHUNK)])
        plsc.subcore_barrier()

        idx_load(0, src0, dstl0, val0, isem0)
        idx_load(1, src1, dstl1, val1, isem1)
        idx_wait(0, src0, dstl0, val0, isem0)
        pltpu.async_copy(ecat.at[src0], rows0, gsem0)

        def piped(i, c2):
            c0 = 2 * i
            pltpu.make_async_copy(ecat.at[src0], rows0, gsem0).wait()
            copy_dst(dstl0, sdst0)
            scale(rows0, val0)

            @pl.when(i > 0)
            def _():
                pltpu.make_async_copy(rows1, accum.at[sdst1], ssem1).wait()

            idx_wait(c0 + 1, src1, dstl1, val1, isem1)
            pltpu.async_copy(ecat.at[src1], rows1, gsem1)

            @pl.when(i < NPIPE - 1)
            def _():
                idx_load(c0 + 2, src0, dstl0, val0, isem0)

            pltpu.async_copy(rows0, accum.at[sdst0], ssem0, add=True)
            pltpu.make_async_copy(ecat.at[src1], rows1, gsem1).wait()
            copy_dst(dstl1, sdst1)
            scale(rows1, val1)
            pltpu.make_async_copy(rows0, accum.at[sdst0], ssem0).wait()

            @pl.when(i < NPIPE - 1)
            def _():
                idx_wait(c0 + 2, src0, dstl0, val0, isem0)
                pltpu.async_copy(ecat.at[src0], rows0, gsem0)
                idx_load(c0 + 3, src1, dstl1, val1, isem1)

            pltpu.async_copy(rows1, accum.at[sdst1], ssem1, add=True)
            return c2

        lax.fori_loop(0, NPIPE, piped, 0)
        pltpu.make_async_copy(rows1, accum.at[sdst1], ssem1).wait()
        plsc.subcore_barrier()
        for k in range(RPT // CHUNK):
            r0 = row0 + k * CHUNK
            pltpu.sync_copy(accum.at[pl.ds(r0, CHUNK)],
                            out.at[p, pl.ds(r0, CHUNK)])
        return carry

    lax.fori_loop(0, 2, run_phase, 0)


_sc_spmm = functools.partial(
    pl.kernel,
    _sc_spmm_body,
    mesh=_sc_mesh,
    out_type=jax.ShapeDtypeStruct((4, NUP, DIM), jnp.float32),
    scratch_types=[
        pltpu.VMEM((CHUNK,), jnp.int32),
        pltpu.VMEM((CHUNK,), jnp.int32),
        pltpu.VMEM((CHUNK,), jnp.int32),
        pltpu.VMEM((CHUNK,), jnp.int32),
        pltpu.VMEM((CHUNK,), jnp.float32),
        pltpu.VMEM((CHUNK,), jnp.float32),
        pltpu.VMEM((CHUNK,), jnp.int32),
        pltpu.VMEM((CHUNK,), jnp.int32),
        pltpu.VMEM((CHUNK, DIM), jnp.float32),
        pltpu.VMEM((CHUNK, DIM), jnp.float32),
        pltpu.VMEM_SHARED((NUP, DIM), jnp.float32),
        pltpu.SemaphoreType.DMA,
        pltpu.SemaphoreType.DMA,
        pltpu.SemaphoreType.DMA,
        pltpu.SemaphoreType.DMA,
        pltpu.SemaphoreType.DMA,
        pltpu.SemaphoreType.DMA,
    ],
)()


GPT = 2 * NB // (NCORE * NSUB)
GC = 128


def _sc_gather_body(ecat, idx, out, idx_v, rows_v, sem):
    cid = lax.axis_index("c")
    sid = lax.axis_index("s")
    wid = sid * NCORE + cid
    base = wid * GPT

    def chunk(i, c):
        off = base + i * GC
        pltpu.sync_copy(idx.at[pl.ds(off, GC)], idx_v)
        pltpu.async_copy(ecat.at[idx_v], rows_v, sem).wait()
        pltpu.sync_copy(rows_v, out.at[pl.ds(off, GC)])
        return c

    lax.fori_loop(0, GPT // GC, chunk, 0)


_sc_gather = functools.partial(
    pl.kernel,
    _sc_gather_body,
    mesh=_sc_mesh,
    out_type=jax.ShapeDtypeStruct((2 * NB, DIM), jnp.float32),
    scratch_types=[
        pltpu.VMEM((GC,), jnp.int32),
        pltpu.VMEM((GC, DIM), jnp.float32),
        pltpu.SemaphoreType.DMA,
    ],
)()


RB = 1000


def _tc_transform_body(x_ref, zp_ref, zn_ref, w_ref, b_ref, o_ref):
    x = x_ref[...]
    zp = zp_ref[0]
    zn = zn_ref[0]
    w = w_ref[0]
    acc = jnp.dot(x, w[0].T, preferred_element_type=jnp.float32)
    acc += jnp.dot(zp, w[1].T, preferred_element_type=jnp.float32)
    acc += jnp.dot(zp * x, w[2].T, preferred_element_type=jnp.float32)
    acc += jnp.dot(zn, w[3].T, preferred_element_type=jnp.float32)
    acc += jnp.dot(zn * x, w[4].T, preferred_element_type=jnp.float32)
    acc += b_ref[0]
    o_ref[...] = jnp.where(acc >= 0.0, acc, 0.2 * acc)


def _tc_transform(ecat, zcat, w, b):
    nb = (2 * NU) // RB
    half = nb // 2
    return pl.pallas_call(
        _tc_transform_body,
        grid=(nb,),
        in_specs=[
            pl.BlockSpec((RB, DIM), lambda j: (j, 0)),
            pl.BlockSpec((1, RB, DIM), lambda j: (j // 10, j % 10, 0)),
            pl.BlockSpec((1, RB, DIM), lambda j: (2 + j // 10, j % 10, 0)),
            pl.BlockSpec((1, 5, DIM, DIM), lambda j: (j // 10, 0, 0, 0)),
            pl.BlockSpec((1, 1, DIM), lambda j: (j // 10, 0, 0)),
        ],
        out_specs=pl.BlockSpec((RB, DIM), lambda j: (j, 0)),
        out_shape=jax.ShapeDtypeStruct((2 * NU, DIM), jnp.float32),
    )(ecat, zcat, zcat, w, b)


LB = 1024
NLB = NB // LB


def _tc_loss_body(u_ref, i_ref, y_ref, logit_ref, acc_ref):
    j = pl.program_id(0)
    u = u_ref[...]
    iv = i_ref[...]
    nrm = jnp.sqrt(jnp.sum(u * u, axis=1, keepdims=True))
    un = u / jnp.maximum(nrm, 1e-12)
    lg = jnp.sum(un * iv, axis=1)
    y = y_ref[0, 0, :]
    per = jnp.maximum(lg, 0.0) - lg * y + jnp.log1p(jnp.exp(-jnp.abs(lg)))
    bce = jnp.sum(per)
    reg = jnp.sum(un * un) + jnp.sum(iv * iv)
    lanes = lax.broadcasted_iota(jnp.int32, (1, 128), 1)
    row = jnp.where(lanes == 0, bce, jnp.where(lanes == 1, reg, 0.0))
    logit_ref[0, 0, :] = lg

    @pl.when(j == 0)
    def _():
        acc_ref[...] = row

    @pl.when(j > 0)
    def _():
        acc_ref[...] += row


def _tc_loss(rows, y3):
    return pl.pallas_call(
        _tc_loss_body,
        grid=(NLB,),
        in_specs=[
            pl.BlockSpec((LB, DIM), lambda j: (j, 0)),
            pl.BlockSpec((LB, DIM), lambda j: (j + NLB, 0)),
            pl.BlockSpec((1, 1, LB), lambda j: (j, 0, 0)),
        ],
        out_specs=[
            pl.BlockSpec((1, 1, LB), lambda j: (j, 0, 0)),
            pl.BlockSpec((1, 128), lambda j: (0, 0)),
        ],
        out_shape=[
            jax.ShapeDtypeStruct((NLB, 1, LB), jnp.float32),
            jax.ShapeDtypeStruct((1, 128), jnp.float32),
        ],
    )(rows, rows, y3)


def kernel(E_u_0, E_i_0, Wu, bu, Wi, bi, pos_values, neg_values,
           pos_edge_index, neg_edge_index, uids, iids, labels):
    pr = pos_edge_index[0].astype(jnp.int32)
    pc = pos_edge_index[1].astype(jnp.int32)
    nr = neg_edge_index[0].astype(jnp.int32)
    nc = neg_edge_index[1].astype(jnp.int32)
    def _stage(x, fill):
        x4 = x.reshape(4, NSUB, EPT)
        pad = jnp.full((4, NSUB, EPTP - EPT), fill, x.dtype)
        return jnp.concatenate([x4, pad], axis=2).reshape(4 * NSUB * EPTP)

    srcs = _stage(jnp.concatenate([pc + NU, pr, nc + NU, nr]), 0)
    dsts = _stage(jnp.concatenate([pr, pc, nr, nc]), 0)
    vals = _stage(jnp.concatenate([pos_values, pos_values, neg_values,
                                   neg_values]), 0.0)
    wall = jnp.stack([Wu, Wi], axis=1)
    ball = jnp.stack([bu.sum(axis=1), bi.sum(axis=1)], axis=1)
    ball = ball.reshape(NLAYER, 2, 1, DIM)
    ecat = jnp.concatenate([E_u_0, E_i_0], axis=0)

    for l in range(NLAYER):
        zcat = _sc_spmm(ecat, srcs, dsts, vals)
        ecat = _tc_transform(ecat, zcat, wall[l], ball[l])

    gidx = jnp.concatenate([uids.astype(jnp.int32),
                            iids.astype(jnp.int32) + NU])
    rows = _sc_gather(ecat, gidx)
    y3 = labels.astype(jnp.float32).reshape(NLB, 1, LB)
    logit3, acc = _tc_loss(rows, y3)
    logits = logit3.reshape(NB)
    loss = acc[0, 0] / NB + 1e-6 * acc[0, 1]
    return (loss, logits)


# speedup vs baseline: 1.1037x; 1.1037x over previous
import functools

import jax
import jax.numpy as jnp
from jax import lax
from jax.experimental import pallas as pl
from jax.experimental.pallas import tpu as pltpu
from jax.experimental.pallas import tpu_sc as plsc

NU = 10000
NI = 10000
DIM = 128
NLAYER = 3
NE = 320000
NB = 16384

NCORE = 2
NSUB = 16
LANE = 16

CHUNK = 64
EPT = NE // NSUB
NCHUNKP = 320
EPTP = NCHUNKP * CHUNK
NPIPE = NCHUNKP // 2
NUP = 10240
RPT = NUP // NSUB

_sc_mesh = plsc.VectorSubcoreMesh(core_axis_name="c", subcore_axis_name="s")


NDEP = 4
NQUAD = NCHUNKP // NDEP


def _sc_spmm_body(ecat, srcs, dsts, vals, out, *rest):
    src_b = rest[0:NDEP]
    dstl_b = rest[NDEP:2 * NDEP]
    val_b = rest[2 * NDEP:3 * NDEP]
    sdst_b = rest[3 * NDEP:4 * NDEP]
    rows_b = rest[4 * NDEP:5 * NDEP]
    accum = rest[5 * NDEP]
    isem_b = rest[5 * NDEP + 1:5 * NDEP + 1 + NDEP]
    gsem_b = rest[5 * NDEP + 1 + NDEP:5 * NDEP + 1 + 2 * NDEP]
    ssem_b = rest[5 * NDEP + 1 + 2 * NDEP:5 * NDEP + 1 + 3 * NDEP]
    cid = lax.axis_index("c")
    sid = lax.axis_index("s")
    row0 = sid * RPT
    zv = jnp.zeros((LANE,), jnp.float32)

    def scale(buf, valb):
        def grp(g, cc):
            base = g * LANE
            vv = valb[pl.ds(base, LANE)]
            for t in range(LANE):
                bv = vv[t]
                for j in range(DIM // LANE):
                    sl = pl.ds(j * LANE, LANE)
                    buf[base + t, sl] = buf[base + t, sl] * bv
            return cc

        lax.fori_loop(0, CHUNK // LANE, grp, 0)

    def copy_dst(dl, sd):
        for g in range(CHUNK // LANE):
            sl = pl.ds(g * LANE, LANE)
            sd[sl] = dl[sl]

    def run_phase(s, carry):
        p = cid * 2 + s
        ebase = (p * NSUB + sid) * EPTP

        def idx_load(c, b):
            off = ebase + c * CHUNK
            pltpu.async_copy(srcs.at[pl.ds(off, CHUNK)], src_b[b], isem_b[b])
            pltpu.async_copy(dsts.at[pl.ds(off, CHUNK)], dstl_b[b], isem_b[b])
            pltpu.async_copy(vals.at[pl.ds(off, CHUNK)], val_b[b], isem_b[b])

        def idx_wait(c, b):
            off = ebase + c * CHUNK
            pltpu.make_async_copy(srcs.at[pl.ds(off, CHUNK)], src_b[b],
                                  isem_b[b]).wait()
            pltpu.make_async_copy(dsts.at[pl.ds(off, CHUNK)], dstl_b[b],
                                  isem_b[b]).wait()
            pltpu.make_async_copy(vals.at[pl.ds(off, CHUNK)], val_b[b],
                                  isem_b[b]).wait()

        def zrow(r, cc):
            for j in range(DIM // LANE):
                rows_b[0][r, pl.ds(j * LANE, LANE)] = zv
            return cc

        lax.fori_loop(0, CHUNK, zrow, 0)
        for k in range(RPT // CHUNK):
            pltpu.sync_copy(rows_b[0],
                            accum.at[pl.ds(row0 + k * CHUNK, CHUNK)])
        plsc.subcore_barrier()

        for b in range(NDEP):
            idx_load(b, b)
        idx_wait(0, 0)
        pltpu.async_copy(ecat.at[src_b[0]], rows_b[0], gsem_b[0])
        idx_wait(1, 1)
        pltpu.async_copy(ecat.at[src_b[1]], rows_b[1], gsem_b[1])

        def piped(q, c2):
            for b in range(NDEP):
                c = NDEP * q + b
                b2 = (b + 2) % NDEP
                pltpu.make_async_copy(ecat.at[src_b[b]], rows_b[b],
                                      gsem_b[b]).wait()
                copy_dst(dstl_b[b], sdst_b[b])
                scale(rows_b[b], val_b[b])
                pltpu.async_copy(rows_b[b], accum.at[sdst_b[b]], ssem_b[b],
                                 add=True)

                @pl.when(q < NQUAD - 1)
                def _():
                    idx_load(c + NDEP, b)

                def _wait_scatter():
                    pltpu.make_async_copy(rows_b[b2], accum.at[sdst_b[b2]],
                                          ssem_b[b2]).wait()

                def _launch_gather():
                    idx_wait(c + 2, b2)
                    pltpu.async_copy(ecat.at[src_b[b2]], rows_b[b2],
                                     gsem_b[b2])

                if b < 2:
                    @pl.when(q > 0)
                    def _():
                        _wait_scatter()

                    _launch_gather()
                else:
                    _wait_scatter()

                    @pl.when(q < NQUAD - 1)
                    def _():
                        _launch_gather()
            return c2

        lax.fori_loop(0, NQUAD, piped, 0)
        pltpu.make_async_copy(rows_b[2], accum.at[sdst_b[2]], ssem_b[2]).wait()
        pltpu.make_async_copy(rows_b[3], accum.at[sdst_b[3]], ssem_b[3]).wait()
        plsc.subcore_barrier()
        for k in range(RPT // CHUNK):
            r0 = row0 + k * CHUNK
            pltpu.sync_copy(accum.at[pl.ds(r0, CHUNK)],
                            out.at[p, pl.ds(r0, CHUNK)])
        return carry

    lax.fori_loop(0, 2, run_phase, 0)


_sc_spmm = functools.partial(
    pl.kernel,
    _sc_spmm_body,
    mesh=_sc_mesh,
    out_type=jax.ShapeDtypeStruct((4, NUP, DIM), jnp.float32),
    scratch_types=(
        [pltpu.VMEM((CHUNK,), jnp.int32) for _ in range(NDEP)]
        + [pltpu.VMEM((CHUNK,), jnp.int32) for _ in range(NDEP)]
        + [pltpu.VMEM((CHUNK,), jnp.float32) for _ in range(NDEP)]
        + [pltpu.VMEM((CHUNK,), jnp.int32) for _ in range(NDEP)]
        + [pltpu.VMEM((CHUNK, DIM), jnp.float32) for _ in range(NDEP)]
        + [pltpu.VMEM_SHARED((NUP, DIM), jnp.float32)]
        + [pltpu.SemaphoreType.DMA for _ in range(3 * NDEP)]
    ),
)()


GPT = 2 * NB // (NCORE * NSUB)
GC = 128


def _sc_gather_body(ecat, idx, out, idx_v, rows_v, sem):
    cid = lax.axis_index("c")
    sid = lax.axis_index("s")
    wid = sid * NCORE + cid
    base = wid * GPT

    def chunk(i, c):
        off = base + i * GC
        pltpu.sync_copy(idx.at[pl.ds(off, GC)], idx_v)
        pltpu.async_copy(ecat.at[idx_v], rows_v, sem).wait()
        pltpu.sync_copy(rows_v, out.at[pl.ds(off, GC)])
        return c

    lax.fori_loop(0, GPT // GC, chunk, 0)


_sc_gather = functools.partial(
    pl.kernel,
    _sc_gather_body,
    mesh=_sc_mesh,
    out_type=jax.ShapeDtypeStruct((2 * NB, DIM), jnp.float32),
    scratch_types=[
        pltpu.VMEM((GC,), jnp.int32),
        pltpu.VMEM((GC, DIM), jnp.float32),
        pltpu.SemaphoreType.DMA,
    ],
)()


RB = 1000


def _tc_transform_body(x_ref, zp_ref, zn_ref, w_ref, b_ref, o_ref):
    x = x_ref[...]
    zp = zp_ref[0]
    zn = zn_ref[0]
    w = w_ref[0]
    acc = jnp.dot(x, w[0].T, preferred_element_type=jnp.float32)
    acc += jnp.dot(zp, w[1].T, preferred_element_type=jnp.float32)
    acc += jnp.dot(zp * x, w[2].T, preferred_element_type=jnp.float32)
    acc += jnp.dot(zn, w[3].T, preferred_element_type=jnp.float32)
    acc += jnp.dot(zn * x, w[4].T, preferred_element_type=jnp.float32)
    acc += b_ref[0]
    o_ref[...] = jnp.where(acc >= 0.0, acc, 0.2 * acc)


def _tc_transform(ecat, zcat, w, b):
    nb = (2 * NU) // RB
    half = nb // 2
    return pl.pallas_call(
        _tc_transform_body,
        grid=(nb,),
        in_specs=[
            pl.BlockSpec((RB, DIM), lambda j: (j, 0)),
            pl.BlockSpec((1, RB, DIM), lambda j: (j // 10, j % 10, 0)),
            pl.BlockSpec((1, RB, DIM), lambda j: (2 + j // 10, j % 10, 0)),
            pl.BlockSpec((1, 5, DIM, DIM), lambda j: (j // 10, 0, 0, 0)),
            pl.BlockSpec((1, 1, DIM), lambda j: (j // 10, 0, 0)),
        ],
        out_specs=pl.BlockSpec((RB, DIM), lambda j: (j, 0)),
        out_shape=jax.ShapeDtypeStruct((2 * NU, DIM), jnp.float32),
    )(ecat, zcat, zcat, w, b)


LB = 1024
NLB = NB // LB


def _tc_loss_body(u_ref, i_ref, y_ref, logit_ref, acc_ref):
    j = pl.program_id(0)
    u = u_ref[...]
    iv = i_ref[...]
    nrm = jnp.sqrt(jnp.sum(u * u, axis=1, keepdims=True))
    un = u / jnp.maximum(nrm, 1e-12)
    lg = jnp.sum(un * iv, axis=1)
    y = y_ref[0, 0, :]
    per = jnp.maximum(lg, 0.0) - lg * y + jnp.log1p(jnp.exp(-jnp.abs(lg)))
    bce = jnp.sum(per)
    reg = jnp.sum(un * un) + jnp.sum(iv * iv)
    lanes = lax.broadcasted_iota(jnp.int32, (1, 128), 1)
    row = jnp.where(lanes == 0, bce, jnp.where(lanes == 1, reg, 0.0))
    logit_ref[0, 0, :] = lg

    @pl.when(j == 0)
    def _():
        acc_ref[...] = row

    @pl.when(j > 0)
    def _():
        acc_ref[...] += row


def _tc_loss(rows, y3):
    return pl.pallas_call(
        _tc_loss_body,
        grid=(NLB,),
        in_specs=[
            pl.BlockSpec((LB, DIM), lambda j: (j, 0)),
            pl.BlockSpec((LB, DIM), lambda j: (j + NLB, 0)),
            pl.BlockSpec((1, 1, LB), lambda j: (j, 0, 0)),
        ],
        out_specs=[
            pl.BlockSpec((1, 1, LB), lambda j: (j, 0, 0)),
            pl.BlockSpec((1, 128), lambda j: (0, 0)),
        ],
        out_shape=[
            jax.ShapeDtypeStruct((NLB, 1, LB), jnp.float32),
            jax.ShapeDtypeStruct((1, 128), jnp.float32),
        ],
    )(rows, rows, y3)


def kernel(E_u_0, E_i_0, Wu, bu, Wi, bi, pos_values, neg_values,
           pos_edge_index, neg_edge_index, uids, iids, labels):
    pr = pos_edge_index[0].astype(jnp.int32)
    pc = pos_edge_index[1].astype(jnp.int32)
    nr = neg_edge_index[0].astype(jnp.int32)
    nc = neg_edge_index[1].astype(jnp.int32)
    def _stage(x, fill):
        x4 = x.reshape(4, NSUB, EPT)
        pad = jnp.full((4, NSUB, EPTP - EPT), fill, x.dtype)
        return jnp.concatenate([x4, pad], axis=2).reshape(4 * NSUB * EPTP)

    srcs = _stage(jnp.concatenate([pc + NU, pr, nc + NU, nr]), 0)
    dsts = _stage(jnp.concatenate([pr, pc, nr, nc]), 0)
    vals = _stage(jnp.concatenate([pos_values, pos_values, neg_values,
                                   neg_values]), 0.0)
    wall = jnp.stack([Wu, Wi], axis=1)
    ball = jnp.stack([bu.sum(axis=1), bi.sum(axis=1)], axis=1)
    ball = ball.reshape(NLAYER, 2, 1, DIM)
    ecat = jnp.concatenate([E_u_0, E_i_0], axis=0)

    for l in range(NLAYER):
        zcat = _sc_spmm(ecat, srcs, dsts, vals)
        ecat = _tc_transform(ecat, zcat, wall[l], ball[l])

    gidx = jnp.concatenate([uids.astype(jnp.int32),
                            iids.astype(jnp.int32) + NU])
    rows = _sc_gather(ecat, gidx)
    y3 = labels.astype(jnp.float32).reshape(NLB, 1, LB)
    logit3, acc = _tc_loss(rows, y3)
    logits = logit3.reshape(NB)
    loss = acc[0, 0] / NB + 1e-6 * acc[0, 1]
    return (loss, logits)
